# baseline (speedup 1.0000x reference)
import functools
import math

import jax
import jax.numpy as jnp
import numpy as np
from jax import lax
from jax.experimental import pallas as pl
from jax.experimental.pallas import tpu as pltpu
from jax.experimental.pallas import tpu_sc as plsc

_D = 1024
_SEQ = 4096
_BATCH = 4
_B = _BATCH * _SEQ
_NW = 32
_BPW = _B // _NW
_R = 16
_NCHUNK = _BPW // _R
_LANES = 16
_VPR = _D // _LANES
_SCALE = math.sqrt(_D)


def _pe_np(seq_len: int, d_model: int) -> np.ndarray:
    pos = np.arange(seq_len, dtype=np.float32)[:, None]
    div = np.exp(
        np.arange(0, d_model, 2, dtype=np.float32) * (-math.log(10000.0) / d_model)
    )
    pe = np.zeros((seq_len, d_model), dtype=np.float32)
    pe[:, 0::2] = np.sin(pos * div)
    pe[:, 1::2] = np.cos(pos * div)
    return pe


_PE = _pe_np(_SEQ, _D)

_mesh = plsc.VectorSubcoreMesh(core_axis_name="c", subcore_axis_name="s")


@functools.partial(
    pl.kernel,
    out_type=jax.ShapeDtypeStruct((_B, _D), jnp.float32),
    mesh=_mesh,
    scratch_types=[
        pltpu.VMEM((_BPW,), jnp.int32),
        pltpu.VMEM((_R, _D), jnp.float32),
        pltpu.VMEM((_R, _D), jnp.float32),
        pltpu.SemaphoreType.DMA,
    ],
)
def _emb(idx_hbm, table_hbm, pe_hbm, out_hbm, idx_v, rows_v, pe_v, sem):
    c = lax.axis_index("c")
    s = lax.axis_index("s")
    wid = s * 2 + c
    base = wid * _BPW
    pe_base = lax.rem(wid, _NW // _BATCH) * _BPW
    pltpu.sync_copy(idx_hbm.at[pl.ds(base, _BPW)], idx_v)

    def chunk(g, carry):
        off = pl.multiple_of(g * _R, _R)
        iv = idx_v[pl.ds(off, _LANES)]
        pltpu.async_copy(table_hbm.at[iv], rows_v, sem).wait()
        pltpu.sync_copy(pe_hbm.at[pl.ds(pe_base + off, _R)], pe_v)

        def row(i, c2):
            for j in range(_VPR):
                sl = pl.ds(j * _LANES, _LANES)
                rows_v[i, sl] = rows_v[i, sl] * _SCALE + pe_v[i, sl]
            return c2

        lax.fori_loop(0, _R, row, 0)
        pltpu.sync_copy(rows_v, out_hbm.at[pl.ds(base + off, _R)])
        return carry

    lax.fori_loop(0, _NCHUNK, chunk, 0)


def kernel(x, table):
    idx = x.reshape(_B)
    pe = jnp.asarray(_PE)
    out = _emb(idx, table, pe)
    return out.reshape(_BATCH, _SEQ, _D)


# speedup vs baseline: 1.5414x; 1.5414x over previous
import functools
import math

import jax
import jax.numpy as jnp
import numpy as np
from jax import lax
from jax.experimental import pallas as pl
from jax.experimental.pallas import tpu as pltpu
from jax.experimental.pallas import tpu_sc as plsc

_D = 1024
_SEQ = 4096
_BATCH = 4
_B = _BATCH * _SEQ
_NW = 32
_PPW = _SEQ // _NW
_CP = 4
_RPC = _CP * _BATCH
_CH = _PPW // _CP
_LANES = 16
_VPR = _D // _LANES
_SCALE = math.sqrt(_D)


def _pe_np(seq_len: int, d_model: int) -> np.ndarray:
    pos = np.arange(seq_len, dtype=np.float32)[:, None]
    div = np.exp(
        np.arange(0, d_model, 2, dtype=np.float32) * (-math.log(10000.0) / d_model)
    )
    pe = np.zeros((seq_len, d_model), dtype=np.float32)
    pe[:, 0::2] = np.sin(pos * div)
    pe[:, 1::2] = np.cos(pos * div)
    return pe


_PE = _pe_np(_SEQ, _D)

_mesh = plsc.VectorSubcoreMesh(core_axis_name="c", subcore_axis_name="s")


@functools.partial(
    pl.kernel,
    out_type=jax.ShapeDtypeStruct((_B, _D), jnp.float32),
    mesh=_mesh,
    scratch_types=[
        pltpu.VMEM((_BATCH * _PPW,), jnp.int32),
        pltpu.VMEM((2, _RPC, _D), jnp.float32),
        pltpu.VMEM((2, _CP, _D), jnp.float32),
        pltpu.VMEM((2, _RPC, _D), jnp.float32),
        pltpu.SemaphoreType.DMA,
        pltpu.SemaphoreType.DMA,
        pltpu.SemaphoreType.DMA,
        pltpu.SemaphoreType.DMA,
    ],
)
def _emb(idx_hbm, table_hbm, pe_hbm, out_hbm,
         idx_v, rows_v, pe_v, res_v, in_sem0, in_sem1, out_sem0, out_sem1):
    c = lax.axis_index("c")
    s = lax.axis_index("s")
    wid = s * 2 + c
    pos0 = wid * _PPW

    pltpu.sync_copy(idx_hbm.at[pl.ds(wid * _BATCH * _PPW, _BATCH * _PPW)],
                    idx_v)

    in_sems = (in_sem0, in_sem1)
    out_sems = (out_sem0, out_sem1)

    def issue_in(g, slot):
        iv = idx_v.at[pl.ds(g * _RPC, _RPC)]
        pltpu.async_copy(table_hbm.at[iv], rows_v.at[slot], in_sems[slot])
        pltpu.async_copy(pe_hbm.at[pl.ds(pos0 + g * _CP, _CP)],
                         pe_v.at[slot], in_sems[slot])

    issue_in(0, 0)
    issue_in(1, 1)

    def body(gg, carry):
        for slot in range(2):
            g = gg * 2 + slot
            iv = idx_v.at[pl.ds(g * _RPC, _RPC)]
            pltpu.make_async_copy(table_hbm.at[iv], rows_v.at[slot],
                                  in_sems[slot]).wait()
            pltpu.make_async_copy(pe_hbm.at[pl.ds(0, _CP)], pe_v.at[slot],
                                  in_sems[slot]).wait()

            @pl.when(gg >= 1)
            def _():
                pltpu.make_async_copy(res_v.at[slot],
                                      out_hbm.at[pl.ds(0, _RPC)],
                                      out_sems[slot]).wait()

            def jbody(j, c2):
                off = pl.multiple_of(j * _LANES, _LANES)
                sl = pl.ds(off, _LANES)
                for p in range(_CP):
                    pe_r = pe_v[slot, p, sl]
                    for b in range(_BATCH):
                        i = b * _CP + p
                        res_v[slot, i, sl] = rows_v[slot, i, sl] * _SCALE + pe_r
                return c2

            lax.fori_loop(0, _VPR, jbody, 0)

            for b in range(_BATCH):
                pltpu.async_copy(
                    res_v.at[slot, pl.ds(b * _CP, _CP)],
                    out_hbm.at[pl.ds(b * _SEQ + pos0 + g * _CP, _CP)],
                    out_sems[slot])

            @pl.when(gg < _CH // 2 - 1)
            def _():
                issue_in(g + 2, slot)
        return carry

    lax.fori_loop(0, _CH // 2, body, 0)

    for slot in range(2):
        pltpu.make_async_copy(res_v.at[slot], out_hbm.at[pl.ds(0, _RPC)],
                              out_sems[slot]).wait()


def kernel(x, table):
    idx = (x.reshape(_BATCH, _NW, _CH, _CP)
            .transpose(1, 2, 0, 3)
            .reshape(_B))
    pe = jnp.asarray(_PE)
    out = _emb(idx, table, pe)
    return out.reshape(_BATCH, _SEQ, _D)


# speedup vs baseline: 1.6525x; 1.0721x over previous
import functools
import math

import jax
import jax.numpy as jnp
import numpy as np
from jax import lax
from jax.experimental import pallas as pl
from jax.experimental.pallas import tpu as pltpu
from jax.experimental.pallas import tpu_sc as plsc

_D = 1024
_SEQ = 4096
_BATCH = 4
_B = _BATCH * _SEQ
_NW = 32
_BPW = _B // _NW
_R = 16
_CH = _BPW // _R
_NSLOT = 4
_SCALE = math.sqrt(_D)
_BR = 512
_PB = _SEQ // _BR


def _pe_np(seq_len: int, d_model: int) -> np.ndarray:
    pos = np.arange(seq_len, dtype=np.float32)[:, None]
    div = np.exp(
        np.arange(0, d_model, 2, dtype=np.float32) * (-math.log(10000.0) / d_model)
    )
    pe = np.zeros((seq_len, d_model), dtype=np.float32)
    pe[:, 0::2] = np.sin(pos * div)
    pe[:, 1::2] = np.cos(pos * div)
    return pe


_PE = _pe_np(_SEQ, _D)

_mesh = plsc.VectorSubcoreMesh(core_axis_name="c", subcore_axis_name="s")


@functools.partial(
    pl.kernel,
    out_type=jax.ShapeDtypeStruct((_B, _D), jnp.float32),
    mesh=_mesh,
    scratch_types=[
        pltpu.VMEM((_BPW,), jnp.int32),
        pltpu.VMEM((_NSLOT, _R, _D), jnp.float32),
        pltpu.SemaphoreType.DMA,
        pltpu.SemaphoreType.DMA,
        pltpu.SemaphoreType.DMA,
        pltpu.SemaphoreType.DMA,
        pltpu.SemaphoreType.DMA,
        pltpu.SemaphoreType.DMA,
        pltpu.SemaphoreType.DMA,
        pltpu.SemaphoreType.DMA,
    ],
)
def _gather(idx_hbm, table_hbm, out_hbm, idx_v, rows_v,
            isem0, isem1, isem2, isem3, osem0, osem1, osem2, osem3):
    c = lax.axis_index("c")
    s = lax.axis_index("s")
    wid = s * 2 + c
    base = wid * _BPW
    in_sems = (isem0, isem1, isem2, isem3)
    out_sems = (osem0, osem1, osem2, osem3)

    pltpu.sync_copy(idx_hbm.at[pl.ds(base, _BPW)], idx_v)

    def issue_in(g, slot):
        iv = idx_v.at[pl.ds(g * _R, _R)]
        pltpu.async_copy(table_hbm.at[iv], rows_v.at[slot], in_sems[slot])

    for g0 in range(_NSLOT - 1):
        issue_in(g0, g0)

    def body(q, carry):
        for slot in range(_NSLOT):
            g = q * _NSLOT + slot
            iv = idx_v.at[pl.ds(g * _R, _R)]
            pltpu.make_async_copy(table_hbm.at[iv], rows_v.at[slot],
                                  in_sems[slot]).wait()
            pltpu.async_copy(rows_v.at[slot],
                             out_hbm.at[pl.ds(base + g * _R, _R)],
                             out_sems[slot])

            prev_slot = (slot + _NSLOT - 1) % _NSLOT
            @pl.when(g >= 1)
            def _():
                pltpu.make_async_copy(rows_v.at[prev_slot],
                                      out_hbm.at[pl.ds(0, _R)],
                                      out_sems[prev_slot]).wait()

            @pl.when(g + _NSLOT - 1 < _CH)
            def _():
                issue_in(g + _NSLOT - 1, prev_slot)
        return carry

    lax.fori_loop(0, _CH // _NSLOT, body, 0)

    last_slot = (_CH - 1) % _NSLOT
    pltpu.make_async_copy(rows_v.at[last_slot], out_hbm.at[pl.ds(0, _R)],
                          out_sems[last_slot]).wait()


def _fuse_body(tmp_ref, pe_ref, out_ref):
    out_ref[...] = tmp_ref[...] * _SCALE + pe_ref[...]


_fuse = pl.pallas_call(
    _fuse_body,
    out_shape=jax.ShapeDtypeStruct((_B, _D), jnp.float32),
    grid=(_PB, _BATCH),
    in_specs=[
        pl.BlockSpec((_BR, _D), lambda i, b: (b * _PB + i, 0)),
        pl.BlockSpec((_BR, _D), lambda i, b: (i, 0)),
    ],
    out_specs=pl.BlockSpec((_BR, _D), lambda i, b: (b * _PB + i, 0)),
)


def kernel(x, table):
    idx = x.reshape(_B)
    tmp = _gather(idx, table)
    pe = jnp.asarray(_PE)
    out = _fuse(tmp, pe)
    return out.reshape(_BATCH, _SEQ, _D)


# speedup vs baseline: 1.7360x; 1.0505x over previous
import functools
import math

import jax
import jax.numpy as jnp
import numpy as np
from jax import lax
from jax.experimental import pallas as pl
from jax.experimental.pallas import tpu as pltpu
from jax.experimental.pallas import tpu_sc as plsc

_D = 1024
_SEQ = 4096
_BATCH = 4
_B = _BATCH * _SEQ
_NW = 32
_BPW = _B // _NW
_R = 16
_CH = _BPW // _R
_NSLOT = 4
_SCALE = math.sqrt(_D)
_BR = 1024
_PB = _SEQ // _BR


def _pe_np(seq_len: int, d_model: int) -> np.ndarray:
    pos = np.arange(seq_len, dtype=np.float32)[:, None]
    div = np.exp(
        np.arange(0, d_model, 2, dtype=np.float32) * (-math.log(10000.0) / d_model)
    )
    pe = np.zeros((seq_len, d_model), dtype=np.float32)
    pe[:, 0::2] = np.sin(pos * div)
    pe[:, 1::2] = np.cos(pos * div)
    return pe


_PE = _pe_np(_SEQ, _D)

_mesh = plsc.VectorSubcoreMesh(core_axis_name="c", subcore_axis_name="s")


@functools.partial(
    pl.kernel,
    out_type=jax.ShapeDtypeStruct((_B, _D), jnp.float32),
    mesh=_mesh,
    scratch_types=[
        pltpu.VMEM((_BPW,), jnp.int32),
        pltpu.VMEM((_NSLOT, _R, _D), jnp.float32),
        pltpu.SemaphoreType.DMA,
        pltpu.SemaphoreType.DMA,
        pltpu.SemaphoreType.DMA,
        pltpu.SemaphoreType.DMA,
        pltpu.SemaphoreType.DMA,
        pltpu.SemaphoreType.DMA,
        pltpu.SemaphoreType.DMA,
        pltpu.SemaphoreType.DMA,
    ],
)
def _gather(idx_hbm, table_hbm, out_hbm, idx_v, rows_v,
            isem0, isem1, isem2, isem3, osem0, osem1, osem2, osem3):
    c = lax.axis_index("c")
    s = lax.axis_index("s")
    wid = s * 2 + c
    base = wid * _BPW
    in_sems = (isem0, isem1, isem2, isem3)
    out_sems = (osem0, osem1, osem2, osem3)

    pltpu.sync_copy(idx_hbm.at[pl.ds(base, _BPW)], idx_v)

    def issue_in(g, slot):
        iv = idx_v.at[pl.ds(g * _R, _R)]
        pltpu.async_copy(table_hbm.at[iv], rows_v.at[slot], in_sems[slot])

    for g0 in range(_NSLOT - 1):
        issue_in(g0, g0)

    def body(q, carry):
        for slot in range(_NSLOT):
            g = q * _NSLOT + slot
            iv = idx_v.at[pl.ds(g * _R, _R)]
            pltpu.make_async_copy(table_hbm.at[iv], rows_v.at[slot],
                                  in_sems[slot]).wait()
            pltpu.async_copy(rows_v.at[slot],
                             out_hbm.at[pl.ds(base + g * _R, _R)],
                             out_sems[slot])

            prev_slot = (slot + _NSLOT - 1) % _NSLOT
            @pl.when(g >= 1)
            def _():
                pltpu.make_async_copy(rows_v.at[prev_slot],
                                      out_hbm.at[pl.ds(0, _R)],
                                      out_sems[prev_slot]).wait()

            @pl.when(g + _NSLOT - 1 < _CH)
            def _():
                issue_in(g + _NSLOT - 1, prev_slot)
        return carry

    lax.fori_loop(0, _CH // _NSLOT, body, 0)

    last_slot = (_CH - 1) % _NSLOT
    pltpu.make_async_copy(rows_v.at[last_slot], out_hbm.at[pl.ds(0, _R)],
                          out_sems[last_slot]).wait()


def _fuse_body(tmp_ref, pe_ref, out_ref):
    out_ref[...] = tmp_ref[...] * _SCALE + pe_ref[...]


_fuse = pl.pallas_call(
    _fuse_body,
    out_shape=jax.ShapeDtypeStruct((_B, _D), jnp.float32),
    grid=(_PB, _BATCH),
    in_specs=[
        pl.BlockSpec((_BR, _D), lambda i, b: (b * _PB + i, 0)),
        pl.BlockSpec((_BR, _D), lambda i, b: (i, 0)),
    ],
    out_specs=pl.BlockSpec((_BR, _D), lambda i, b: (b * _PB + i, 0)),
)


def kernel(x, table):
    idx = x.reshape(_B)
    tmp = _gather(idx, table)
    pe = jnp.asarray(_PE)
    out = _fuse(tmp, pe)
    return out.reshape(_BATCH, _SEQ, _D)


# speedup vs baseline: 1.7866x; 1.0291x over previous
import functools
import math

import jax
import jax.numpy as jnp
import numpy as np
from jax import lax
from jax.experimental import pallas as pl
from jax.experimental.pallas import tpu as pltpu
from jax.experimental.pallas import tpu_sc as plsc

_D = 1024
_SEQ = 4096
_BATCH = 4
_B = _BATCH * _SEQ
_NW = 32
_BPW = _B // _NW
_R = 16
_CH = _BPW // _R
_NSLOT = 4
_SCALE = math.sqrt(_D)
_BR = 2048
_PB = _SEQ // _BR


def _pe_np(seq_len: int, d_model: int) -> np.ndarray:
    pos = np.arange(seq_len, dtype=np.float32)[:, None]
    div = np.exp(
        np.arange(0, d_model, 2, dtype=np.float32) * (-math.log(10000.0) / d_model)
    )
    pe = np.zeros((seq_len, d_model), dtype=np.float32)
    pe[:, 0::2] = np.sin(pos * div)
    pe[:, 1::2] = np.cos(pos * div)
    return pe


_PE = _pe_np(_SEQ, _D)

_mesh = plsc.VectorSubcoreMesh(core_axis_name="c", subcore_axis_name="s")


@functools.partial(
    pl.kernel,
    out_type=jax.ShapeDtypeStruct((_B, _D), jnp.float32),
    mesh=_mesh,
    scratch_types=[
        pltpu.VMEM((_BPW,), jnp.int32),
        pltpu.VMEM((_NSLOT, _R, _D), jnp.float32),
        pltpu.SemaphoreType.DMA,
        pltpu.SemaphoreType.DMA,
        pltpu.SemaphoreType.DMA,
        pltpu.SemaphoreType.DMA,
        pltpu.SemaphoreType.DMA,
        pltpu.SemaphoreType.DMA,
        pltpu.SemaphoreType.DMA,
        pltpu.SemaphoreType.DMA,
    ],
)
def _gather(idx_hbm, table_hbm, out_hbm, idx_v, rows_v,
            isem0, isem1, isem2, isem3, osem0, osem1, osem2, osem3):
    c = lax.axis_index("c")
    s = lax.axis_index("s")
    wid = s * 2 + c
    base = wid * _BPW
    in_sems = (isem0, isem1, isem2, isem3)
    out_sems = (osem0, osem1, osem2, osem3)

    pltpu.sync_copy(idx_hbm.at[pl.ds(base, _BPW)], idx_v)

    def issue_in(g, slot):
        iv = idx_v.at[pl.ds(g * _R, _R)]
        pltpu.async_copy(table_hbm.at[iv], rows_v.at[slot], in_sems[slot])

    for g0 in range(_NSLOT - 1):
        issue_in(g0, g0)

    def body(q, carry):
        for slot in range(_NSLOT):
            g = q * _NSLOT + slot
            iv = idx_v.at[pl.ds(g * _R, _R)]
            pltpu.make_async_copy(table_hbm.at[iv], rows_v.at[slot],
                                  in_sems[slot]).wait()
            pltpu.async_copy(rows_v.at[slot],
                             out_hbm.at[pl.ds(base + g * _R, _R)],
                             out_sems[slot])

            prev_slot = (slot + _NSLOT - 1) % _NSLOT
            @pl.when(g >= 1)
            def _():
                pltpu.make_async_copy(rows_v.at[prev_slot],
                                      out_hbm.at[pl.ds(0, _R)],
                                      out_sems[prev_slot]).wait()

            @pl.when(g + _NSLOT - 1 < _CH)
            def _():
                issue_in(g + _NSLOT - 1, prev_slot)
        return carry

    lax.fori_loop(0, _CH // _NSLOT, body, 0)

    last_slot = (_CH - 1) % _NSLOT
    pltpu.make_async_copy(rows_v.at[last_slot], out_hbm.at[pl.ds(0, _R)],
                          out_sems[last_slot]).wait()


def _fuse_body(tmp_ref, pe_ref, out_ref):
    out_ref[...] = tmp_ref[...] * _SCALE + pe_ref[...]


_fuse = pl.pallas_call(
    _fuse_body,
    out_shape=jax.ShapeDtypeStruct((_B, _D), jnp.float32),
    grid=(_PB, _BATCH),
    in_specs=[
        pl.BlockSpec((_BR, _D), lambda i, b: (b * _PB + i, 0)),
        pl.BlockSpec((_BR, _D), lambda i, b: (i, 0)),
    ],
    out_specs=pl.BlockSpec((_BR, _D), lambda i, b: (b * _PB + i, 0)),
)


def kernel(x, table):
    idx = x.reshape(_B)
    tmp = _gather(idx, table)
    pe = jnp.asarray(_PE)
    out = _fuse(tmp, pe)
    return out.reshape(_BATCH, _SEQ, _D)
